# initial kernel scaffold (unmeasured)
import jax
import jax.numpy as jnp
from jax import lax
from jax.experimental import pallas as pl
from jax.experimental.pallas import tpu as pltpu

N_DEV = 8


def kernel(x, w_mat):
    m_per, k = x.shape
    _, n = w_mat.shape
    n_per = n // N_DEV
    m = m_per * N_DEV

    def body(x_ref, w_ref, out_ref, w_buf, y_bufs, w_sem, send_sems, recv_sems):
        my = lax.axis_index("i")

        send_rdmas = []
        for j in range(N_DEV):
            blk = (my + j) % N_DEV
            cp = pltpu.make_async_copy(
                w_ref.at[:, pl.ds(blk * n_per, n_per)],
                w_buf,
                w_sem,
            )
            cp.start()
            cp.wait()

            y = jnp.dot(x_ref[...], w_buf[...],
                        preferred_element_type=jnp.float32)
            y = y * jax.nn.sigmoid(y)

            if j == 0:
                out_ref[pl.ds(my * m_per, m_per), :] = y
            else:
                c = j - 1
                slot = c % 2
                if c >= 2:
                    send_rdmas[c - 2].wait_send()
                y_bufs[slot] = y
                rdma = pltpu.make_async_remote_copy(
                    src_ref=y_bufs.at[slot],
                    dst_ref=out_ref.at[pl.ds(my * m_per, m_per), :],
                    send_sem=send_sems.at[slot],
                    recv_sem=recv_sems.at[my],
                    device_id=(blk,),
                    device_id_type=pl.DeviceIdType.MESH,
                )
                rdma.start()
                send_rdmas.append(rdma)

        for r in send_rdmas[-2:]:
            r.wait_send()

        for j in range(1, N_DEV):
            src = (my - j) % N_DEV
            recv = pltpu.make_async_remote_copy(
                src_ref=y_bufs.at[0],
                dst_ref=out_ref.at[pl.ds(src * m_per, m_per), :],
                send_sem=send_sems.at[0],
                recv_sem=recv_sems.at[src],
                device_id=(src,),
                device_id_type=pl.DeviceIdType.MESH,
            )
            recv.wait_recv()

    return pl.pallas_call(
        body,
        out_shape=jax.ShapeDtypeStruct((m, n_per), jnp.float32),
        in_specs=[
            pl.BlockSpec(memory_space=pltpu.VMEM),
            pl.BlockSpec(memory_space=pltpu.ANY),
        ],
        out_specs=pl.BlockSpec(memory_space=pltpu.VMEM),
        scratch_shapes=[
            pltpu.VMEM((k, n_per), jnp.float32),
            pltpu.VMEM((2, m_per, n_per), jnp.float32),
            pltpu.SemaphoreType.DMA,
            pltpu.SemaphoreType.DMA((2,)),
            pltpu.SemaphoreType.DMA((N_DEV,)),
        ],
        compiler_params=pltpu.CompilerParams(
            collective_id=0,
            vmem_limit_bytes=110 * 1024 * 1024,
        ),
    )(x, w_mat)


# baseline (device time: 209259 ns/iter reference)
import jax
import jax.numpy as jnp
from jax import lax
from jax.experimental import pallas as pl
from jax.experimental.pallas import tpu as pltpu

N_DEV = 8


def kernel(x, w_mat):
    m_per, k = x.shape
    _, n = w_mat.shape
    n_per = n // N_DEV
    m = m_per * N_DEV

    def body(x_ref, w_ref, out_ref, w_buf, y_bufs, w_sem, send_sems, recv_sems):
        my = lax.axis_index("i")

        send_rdmas = []
        for j in range(N_DEV):
            blk = (my + j) % N_DEV
            cp = pltpu.make_async_copy(
                w_ref.at[:, pl.ds(blk * n_per, n_per)],
                w_buf,
                w_sem,
            )
            cp.start()
            cp.wait()

            y = jnp.dot(x_ref[...], w_buf[...],
                        preferred_element_type=jnp.float32)
            y = y * jax.nn.sigmoid(y)

            if j == 0:
                out_ref[pl.ds(my * m_per, m_per), :] = y
            else:
                c = j - 1
                slot = c % 2
                if c >= 2:
                    send_rdmas[c - 2].wait_send()
                y_bufs[slot] = y
                rdma = pltpu.make_async_remote_copy(
                    src_ref=y_bufs.at[slot],
                    dst_ref=out_ref.at[pl.ds(my * m_per, m_per), :],
                    send_sem=send_sems.at[slot],
                    recv_sem=recv_sems.at[my],
                    device_id=(blk,),
                    device_id_type=pl.DeviceIdType.MESH,
                )
                rdma.start()
                send_rdmas.append(rdma)

        for r in send_rdmas[-2:]:
            r.wait_send()

        for j in range(1, N_DEV):
            src = (my - j) % N_DEV
            recv = pltpu.make_async_remote_copy(
                src_ref=y_bufs.at[0],
                dst_ref=out_ref.at[pl.ds(src * m_per, m_per), :],
                send_sem=send_sems.at[0],
                recv_sem=recv_sems.at[src],
                device_id=(src,),
                device_id_type=pl.DeviceIdType.MESH,
            )
            recv.wait_recv()

    return pl.pallas_call(
        body,
        out_shape=jax.ShapeDtypeStruct((m, n_per), jnp.float32),
        in_specs=[
            pl.BlockSpec(memory_space=pltpu.VMEM),
            pl.BlockSpec(memory_space=pltpu.MemorySpace.HBM),
        ],
        out_specs=pl.BlockSpec(memory_space=pltpu.VMEM),
        scratch_shapes=[
            pltpu.VMEM((k, n_per), jnp.float32),
            pltpu.VMEM((2, m_per, n_per), jnp.float32),
            pltpu.SemaphoreType.DMA,
            pltpu.SemaphoreType.DMA((2,)),
            pltpu.SemaphoreType.DMA((N_DEV,)),
        ],
        compiler_params=pltpu.CompilerParams(
            vmem_limit_bytes=110 * 1024 * 1024,
        ),
    )(x, w_mat)


# device time: 195394 ns/iter; 1.0710x vs baseline; 1.0710x over previous
import jax
import jax.numpy as jnp
from jax import lax
from jax.experimental import pallas as pl
from jax.experimental.pallas import tpu as pltpu

N_DEV = 8


def kernel(x, w_mat):
    m_per, k = x.shape
    _, n = w_mat.shape
    n_per = n // N_DEV
    m = m_per * N_DEV

    def body(x_ref, w_ref, out_ref, w_bufs, y_bufs, w_sems, send_sems, recv_sems):
        my = lax.axis_index("i")

        def w_copy(j):
            blk = (my + j) % N_DEV
            return pltpu.make_async_copy(
                w_ref.at[:, pl.ds(blk * n_per, n_per)],
                w_bufs.at[j % 2],
                w_sems.at[j % 2],
            )

        w_copy(0).start()

        send_rdmas = []
        for j in range(N_DEV):
            if j + 1 < N_DEV:
                w_copy(j + 1).start()
            w_copy(j).wait()

            y = jnp.dot(x_ref[...], w_bufs[j % 2],
                        preferred_element_type=jnp.float32)
            y = y * jax.nn.sigmoid(y)

            blk = (my + j) % N_DEV
            if j == 0:
                out_ref[pl.ds(my * m_per, m_per), :] = y
            else:
                c = j - 1
                slot = c % 2
                if c >= 2:
                    send_rdmas[c - 2].wait_send()
                y_bufs[slot] = y
                rdma = pltpu.make_async_remote_copy(
                    src_ref=y_bufs.at[slot],
                    dst_ref=out_ref.at[pl.ds(my * m_per, m_per), :],
                    send_sem=send_sems.at[slot],
                    recv_sem=recv_sems.at[my],
                    device_id=(blk,),
                    device_id_type=pl.DeviceIdType.MESH,
                )
                rdma.start()
                send_rdmas.append(rdma)

        for r in send_rdmas[-2:]:
            r.wait_send()

        for j in range(1, N_DEV):
            src = (my - j) % N_DEV
            recv = pltpu.make_async_remote_copy(
                src_ref=y_bufs.at[0],
                dst_ref=out_ref.at[pl.ds(src * m_per, m_per), :],
                send_sem=send_sems.at[0],
                recv_sem=recv_sems.at[src],
                device_id=(src,),
                device_id_type=pl.DeviceIdType.MESH,
            )
            recv.wait_recv()

    return pl.pallas_call(
        body,
        out_shape=jax.ShapeDtypeStruct((m, n_per), jnp.float32),
        in_specs=[
            pl.BlockSpec(memory_space=pltpu.VMEM),
            pl.BlockSpec(memory_space=pltpu.MemorySpace.HBM),
        ],
        out_specs=pl.BlockSpec(memory_space=pltpu.VMEM),
        scratch_shapes=[
            pltpu.VMEM((2, k, n_per), jnp.float32),
            pltpu.VMEM((2, m_per, n_per), jnp.float32),
            pltpu.SemaphoreType.DMA((2,)),
            pltpu.SemaphoreType.DMA((2,)),
            pltpu.SemaphoreType.DMA((N_DEV,)),
        ],
        compiler_params=pltpu.CompilerParams(
            vmem_limit_bytes=110 * 1024 * 1024,
        ),
    )(x, w_mat)


# device time: 176765 ns/iter; 1.1838x vs baseline; 1.1054x over previous
import jax
import jax.numpy as jnp
from jax import lax
from jax.experimental import pallas as pl
from jax.experimental.pallas import tpu as pltpu

N_DEV = 8


def kernel(x, w_mat):
    m_per, k = x.shape
    _, n = w_mat.shape
    n_per = n // N_DEV
    m = m_per * N_DEV

    def body(x_ref, w_ref, out_ref, w_bufs, y_bufs, w_sems, send_sems, recv_sems):
        my = lax.axis_index("i")

        NSUB = 2
        n_sub = n_per // NSUB

        def w_copy(s):
            blk = (my + s // NSUB) % N_DEV
            off = blk * n_per + (s % NSUB) * n_sub
            return pltpu.make_async_copy(
                w_ref.at[:, pl.ds(off, n_sub)],
                w_bufs.at[s % 2],
                w_sems.at[s % 2],
            )

        w_copy(0).start()

        send_rdmas = []
        for s in range(N_DEV * NSUB):
            j, h = s // NSUB, s % NSUB
            if s + 1 < N_DEV * NSUB:
                w_copy(s + 1).start()
            w_copy(s).wait()

            y = jnp.dot(x_ref[...], w_bufs[s % 2],
                        preferred_element_type=jnp.float32)
            y = y * jax.nn.sigmoid(y)

            if j == 0:
                out_ref[pl.ds(my * m_per, m_per),
                        pl.ds(h * n_sub, n_sub)] = y
            else:
                slot = j - 1
                y_bufs[slot, :, pl.ds(h * n_sub, n_sub)] = y
                if h == NSUB - 1:
                    blk = (my + j) % N_DEV
                    rdma = pltpu.make_async_remote_copy(
                        src_ref=y_bufs.at[slot],
                        dst_ref=out_ref.at[pl.ds(my * m_per, m_per), :],
                        send_sem=send_sems.at[slot],
                        recv_sem=recv_sems.at[my],
                        device_id=(blk,),
                        device_id_type=pl.DeviceIdType.MESH,
                    )
                    rdma.start()
                    send_rdmas.append(rdma)

        for r in send_rdmas:
            r.wait_send()

        for j in range(1, N_DEV):
            src = (my - j) % N_DEV
            recv = pltpu.make_async_remote_copy(
                src_ref=y_bufs.at[0],
                dst_ref=out_ref.at[pl.ds(src * m_per, m_per), :],
                send_sem=send_sems.at[0],
                recv_sem=recv_sems.at[src],
                device_id=(src,),
                device_id_type=pl.DeviceIdType.MESH,
            )
            recv.wait_recv()

    return pl.pallas_call(
        body,
        out_shape=jax.ShapeDtypeStruct((m, n_per), jnp.float32),
        in_specs=[
            pl.BlockSpec(memory_space=pltpu.VMEM),
            pl.BlockSpec(memory_space=pltpu.MemorySpace.HBM),
        ],
        out_specs=pl.BlockSpec(memory_space=pltpu.VMEM),
        scratch_shapes=[
            pltpu.VMEM((2, k, n_per // 2), jnp.float32),
            pltpu.VMEM((N_DEV - 1, m_per, n_per), jnp.float32),
            pltpu.SemaphoreType.DMA((2,)),
            pltpu.SemaphoreType.DMA((N_DEV - 1,)),
            pltpu.SemaphoreType.DMA((N_DEV,)),
        ],
        compiler_params=pltpu.CompilerParams(
            vmem_limit_bytes=110 * 1024 * 1024,
        ),
    )(x, w_mat)
